# baseline (speedup 1.0000x reference)
import functools

import jax
import jax.numpy as jnp
from jax import lax
from jax.experimental import pallas as pl
from jax.experimental.pallas import tpu as pltpu
from jax.experimental.pallas import tpu_sc as plsc

_NC = 2
_NS = 16
_NW = _NC * _NS
_LANES = 16
_R = 32


def _tc_body(x_ref, p_ref, o_ref):
    o_ref[...] = x_ref[...] + p_ref[...]


def _make_sc_add(T, C, sc_batch):
    rows_per_w = T // _NW
    nchunk = rows_per_w // _R
    groups_per_row = C // _LANES
    mesh = plsc.VectorSubcoreMesh(core_axis_name="c", subcore_axis_name="s")

    @functools.partial(
        pl.kernel,
        mesh=mesh,
        out_type=jax.ShapeDtypeStruct((1, T, C), jnp.float32),
        scratch_types=[
            pltpu.VMEM((_R, C), jnp.float32),
            pltpu.VMEM((_R, C), jnp.float32),
        ],
    )
    def sc_add(x_hbm, pos_hbm, out_hbm, xbuf, pbuf):
        wid = lax.axis_index("s") * _NC + lax.axis_index("c")
        base = wid * rows_per_w

        def chunk_body(ci, carry):
            row0 = base + ci * _R
            pltpu.sync_copy(x_hbm.at[sc_batch, pl.ds(row0, _R), :], xbuf)
            pltpu.sync_copy(pos_hbm.at[pl.ds(row0, _R), :], pbuf)

            def row_body(r, c2):
                for g in range(groups_per_row):
                    sl = pl.ds(g * _LANES, _LANES)
                    xbuf[r, sl] = xbuf[r, sl] + pbuf[r, sl]
                return c2

            lax.fori_loop(0, _R, row_body, 0)
            pltpu.sync_copy(xbuf, out_hbm.at[0, pl.ds(row0, _R), :])
            return carry

        lax.fori_loop(0, nchunk, chunk_body, 0)

    return sc_add


def kernel(x, pos_emb):
    B, T, C = x.shape
    BT = 2048
    tc_batches = B - 1
    tc_out = pl.pallas_call(
        _tc_body,
        grid=(T // BT, tc_batches),
        in_specs=[
            pl.BlockSpec((1, BT, C), lambda i, j: (j, i, 0)),
            pl.BlockSpec((BT, C), lambda i, j: (i, 0)),
        ],
        out_specs=pl.BlockSpec((1, BT, C), lambda i, j: (j, i, 0)),
        out_shape=jax.ShapeDtypeStruct((tc_batches, T, C), x.dtype),
    )(x, pos_emb)
    sc_out = _make_sc_add(T, C, B - 1)(x, pos_emb)
    return jnp.concatenate([tc_out, sc_out], axis=0)
